# baseline (speedup 1.0000x reference)
import functools

import jax
import jax.numpy as jnp
from jax import lax
from jax.experimental import pallas as pl
from jax.experimental.pallas import tpu as pltpu
from jax.experimental.pallas import tpu_sc as plsc

H, W = 384, 512
FX, FY = 518.8579, 519.4696
HW = H * W
NUM = 100000
PADN = 100096
NROWS = PADN // 128
NS = 15000
NS_PAD = 16384
PER_TILE = NS_PAD // 32
K_DROP = 15000
N_KEEP = 4 * NS - K_DROP
NTILES = 32


def _sample_position_consts():
    skey = jax.random.key(42)
    sk1, sk2, sk3 = jax.random.split(skey, 3)
    ps = [jnp.pad(jax.random.randint(sk, (NS,), 0, NUM), (0, NS_PAD - NS))
          for sk in (sk1, sk2, sk3)]
    rows = []
    for g in range(2):
        for k in range(3):
            rows.append(ps[k] + g * PADN)
    pidx = jnp.stack(rows).astype(jnp.int32)
    return (pidx.reshape(6, NTILES, PER_TILE)
            .transpose(1, 0, 2).reshape(NTILES, 6 * PER_TILE))


def _prepack_body(idx_ref, out_ref):
    for g in range(2):
        y = idx_ref[2 * g]
        x = idx_ref[2 * g + 1]
        out_ref[g] = y * W + x + g * HW


def _prepack(idx4):
    return pl.pallas_call(
        _prepack_body,
        out_shape=jax.ShapeDtypeStruct((2, NROWS, 128), jnp.int32),
    )(idx4)


def _sc_gather(depth_t, packed, pidx):
    mesh = plsc.VectorSubcoreMesh(
        core_axis_name="c", subcore_axis_name="s", num_cores=2, num_subcores=16
    )
    half = 3 * PER_TILE

    @functools.partial(
        pl.kernel,
        out_type=(
            jax.ShapeDtypeStruct((NTILES, 6 * PER_TILE, 4), jnp.float32),
            jax.ShapeDtypeStruct((NTILES, 6 * PER_TILE), jnp.int32),
        ),
        mesh=mesh,
        compiler_params=pltpu.CompilerParams(use_tc_tiling_on_sc=False),
        scratch_types=[
            pltpu.VMEM((6 * PER_TILE,), jnp.int32),
            pltpu.VMEM((6 * PER_TILE,), jnp.int32),
            pltpu.VMEM((6 * PER_TILE, 4), jnp.float32),
            pltpu.SemaphoreType.DMA,
            pltpu.SemaphoreType.DMA,
            pltpu.SemaphoreType.DMA,
            pltpu.SemaphoreType.DMA,
        ],
    )
    def sc_kernel(depth_hbm, packed_hbm, pidx_hbm, d_out, pk_out,
                  pidx_v, pk_v, d_v, sem_a0, sem_a1, sem_b, sem_y):
        wid = lax.axis_index("s") * 2 + lax.axis_index("c")
        pltpu.sync_copy(pidx_hbm.at[wid], pidx_v)
        cp_a = [
            pltpu.async_copy(packed_hbm.at[pidx_v.at[pl.ds(g * half, half)]],
                             pk_v.at[pl.ds(g * half, half)],
                             sem_a0 if g == 0 else sem_a1)
            for g in range(2)
        ]
        cp_b = []
        for g in range(2):
            cp_a[g].wait()
            cp_b.append(pltpu.async_copy(
                depth_hbm.at[pk_v.at[pl.ds(g * half, half)]],
                d_v.at[pl.ds(g * half, half)], sem_b))
        cp_y = pltpu.async_copy(pk_v, pk_out.at[wid], sem_y)
        for cp in cp_b:
            cp.wait()
        pltpu.sync_copy(d_v, d_out.at[wid])
        cp_y.wait()

    return sc_kernel(depth_t, packed, pidx)


def _tc_loss_body(d_ref, pk_ref, out_ref):
    normals = [[None] * 4, [None] * 4]
    for g in range(2):
        yc, xc = [], []
        for k in range(3):
            flat = pk_ref[:, g * 3 + k, :] - g * HW
            yc.append((flat >> 9).astype(jnp.float32) - float(H // 2))
            xc.append((flat & (W - 1)).astype(jnp.float32) - float(W // 2))
        for b in range(4):
            pts = []
            for k in range(3):
                d = d_ref[:, (g * 4 + b) * 3 + k, :]
                ad = jnp.abs(d)
                pts.append((xc[k] * ad * (1.0 / FX), yc[k] * ad * (1.0 / FY), d))
            e12 = tuple(pts[1][c] - pts[0][c] for c in range(3))
            e13 = tuple(pts[2][c] - pts[0][c] for c in range(3))
            nx = e12[1] * e13[2] - e12[2] * e13[1]
            ny = e12[2] * e13[0] - e12[0] * e13[2]
            nz = e12[0] * e13[1] - e12[1] * e13[0]
            norm = jnp.sqrt(nx * nx + ny * ny + nz * nz)
            denom = norm + jnp.where(norm == 0.0, jnp.float32(0.01),
                                     jnp.float32(0.0))
            normals[g][b] = (nx / denom, ny / denom, nz / denom)

    ji = (lax.broadcasted_iota(jnp.int32, (NTILES, PER_TILE), 0) * PER_TILE
          + lax.broadcasted_iota(jnp.int32, (NTILES, PER_TILE), 1))
    valid = ji < NS

    losses = []
    for b in range(4):
        gt_n, dt_n = normals[0][b], normals[1][b]
        losses.append(sum(jnp.abs(gt_n[c] - dt_n[c]) for c in range(3)))

    inf_bits = jnp.int32(0x7F800000)
    mbits = [jnp.where(valid, lax.bitcast_convert_type(v, jnp.int32), inf_bits)
             for v in losses]

    def bisect(_, lohi):
        lo, hi = lohi
        mid = lo + (hi - lo) // 2
        cnt = sum(jnp.sum((mb <= mid).astype(jnp.int32)) for mb in mbits)
        ge = cnt >= K_DROP
        return jnp.where(ge, lo, mid + 1), jnp.where(ge, mid, hi)

    _, t_bits = lax.fori_loop(0, 31, bisect, (jnp.int32(0), inf_bits))
    t = lax.bitcast_convert_type(t_bits, jnp.float32)

    cnt_lt = jnp.float32(0.0)
    sum_lt = jnp.float32(0.0)
    total = jnp.float32(0.0)
    for b in range(4):
        lt = valid & (losses[b] < t)
        cnt_lt = cnt_lt + jnp.sum(lt.astype(jnp.float32))
        sum_lt = sum_lt + jnp.sum(jnp.where(lt, losses[b], 0.0))
        total = total + jnp.sum(jnp.where(valid, losses[b], 0.0))
    dropped = sum_lt + (jnp.float32(K_DROP) - cnt_lt) * t
    out_ref[0, 0] = (total - dropped) * (1.0 / N_KEEP)


def _tc_loss(d24, pk6):
    return pl.pallas_call(
        _tc_loss_body,
        out_shape=jax.ShapeDtypeStruct((1, 1), jnp.float32),
        out_specs=pl.BlockSpec(memory_space=pltpu.SMEM),
    )(d24, pk6)


def kernel(gt_depth, pred_depth, gt_index, pred_index):
    depth_t = jnp.concatenate(
        [gt_depth.reshape(4, HW).T, pred_depth.reshape(4, HW).T]
    )
    idx4 = jnp.concatenate(
        [gt_index.astype(jnp.int32), pred_index.astype(jnp.int32)]
    )
    idx4 = jnp.pad(idx4, ((0, 0), (0, PADN - NUM))).reshape(4, NROWS, 128)
    packed = _prepack(idx4).reshape(2 * PADN)
    d_out, pk_out = _sc_gather(depth_t, packed, _sample_position_consts())
    d24 = (d_out.reshape(NTILES, 2, 3, PER_TILE, 4)
           .transpose(0, 1, 4, 2, 3).reshape(NTILES, 24, PER_TILE))
    out = _tc_loss(d24, pk_out.reshape(NTILES, 6, PER_TILE))
    return out.reshape(())


# speedup vs baseline: 4.7477x; 4.7477x over previous
import functools

import jax
import jax.numpy as jnp
from jax import lax
from jax.experimental import pallas as pl
from jax.experimental.pallas import tpu as pltpu
from jax.experimental.pallas import tpu_sc as plsc

H, W = 384, 512
FX, FY = 518.8579, 519.4696
HW = H * W
NUM = 100000
PADN = 100096
NROWS = PADN // 128
NS = 15000
NS_PAD = 16384
PER_TILE = NS_PAD // 32
K_DROP = 15000
N_KEEP = 4 * NS - K_DROP
NTILES = 32


def _sample_position_consts():
    skey = jax.random.key(42)
    sk1, sk2, sk3 = jax.random.split(skey, 3)
    ps = [jnp.pad(jax.random.randint(sk, (NS,), 0, NUM), (0, NS_PAD - NS))
          for sk in (sk1, sk2, sk3)]
    rows = []
    for g in range(2):
        for k in range(3):
            rows.append(ps[k] + g * PADN)
    pidx = jnp.stack(rows).astype(jnp.int32)
    return (pidx.reshape(6, NTILES, PER_TILE)
            .transpose(1, 0, 2).reshape(NTILES, 6 * PER_TILE))


def _prepack_body(idx_ref, out_ref):
    for g in range(2):
        y = idx_ref[2 * g]
        x = idx_ref[2 * g + 1]
        out_ref[g] = y * W + x + g * HW


def _prepack(idx4):
    return pl.pallas_call(
        _prepack_body,
        out_shape=jax.ShapeDtypeStruct((2, NROWS, 128), jnp.int32),
    )(idx4)


def _sc_gather(depth_flat, packed, pidx):
    mesh = plsc.VectorSubcoreMesh(
        core_axis_name="c", subcore_axis_name="s", num_cores=2, num_subcores=16
    )
    half = 3 * PER_TILE

    @functools.partial(
        pl.kernel,
        out_type=(
            jax.ShapeDtypeStruct((NTILES, 24 * PER_TILE), jnp.float32),
            jax.ShapeDtypeStruct((NTILES, 6 * PER_TILE), jnp.int32),
        ),
        mesh=mesh,
        compiler_params=pltpu.CompilerParams(use_tc_tiling_on_sc=False),
        scratch_types=[
            pltpu.VMEM((6 * PER_TILE,), jnp.int32),
            pltpu.VMEM((6 * PER_TILE,), jnp.int32),
            pltpu.VMEM((24 * PER_TILE,), jnp.int32),
            pltpu.VMEM((24 * PER_TILE,), jnp.float32),
            pltpu.SemaphoreType.DMA,
            pltpu.SemaphoreType.DMA,
            pltpu.SemaphoreType.DMA,
            pltpu.SemaphoreType.DMA,
        ],
    )
    def sc_kernel(depth_hbm, packed_hbm, pidx_hbm, d_out, pk_out,
                  pidx_v, pk_v, addr_v, d_v, sem_a0, sem_a1, sem_b, sem_y):
        wid = lax.axis_index("s") * 2 + lax.axis_index("c")
        pltpu.sync_copy(pidx_hbm.at[wid], pidx_v)
        cp_a = [
            pltpu.async_copy(packed_hbm.at[pidx_v.at[pl.ds(g * half, half)]],
                             pk_v.at[pl.ds(g * half, half)],
                             sem_a0 if g == 0 else sem_a1)
            for g in range(2)
        ]

        def addr_body(g, j):
            for k in range(3):
                pk = pk_v[pl.ds((g * 3 + k) * PER_TILE + j * 16, 16)]
                for b in range(4):
                    addr_v[pl.ds(((g * 4 + b) * 3 + k) * PER_TILE
                                 + j * 16, 16)] = pk + (g * 3 + b) * HW

        cp_b = []
        for g in range(2):
            cp_a[g].wait()
            lax.fori_loop(0, PER_TILE // 16,
                          lambda j, c, g=g: (addr_body(g, j), c)[1], 0)
            for h in range(2):
                sl = pl.ds((g * 2 + h) * 6 * PER_TILE, 6 * PER_TILE)
                cp_b.append(pltpu.async_copy(
                    depth_hbm.at[addr_v.at[sl]], d_v.at[sl], sem_b))
        cp_y = pltpu.async_copy(pk_v, pk_out.at[wid], sem_y)
        for cp in cp_b:
            cp.wait()
        pltpu.sync_copy(d_v, d_out.at[wid])
        cp_y.wait()

    return sc_kernel(depth_flat, packed, pidx)


def _tc_loss_body(d_ref, pk_ref, out_ref):
    normals = [[None] * 4, [None] * 4]
    for g in range(2):
        yc, xc = [], []
        for k in range(3):
            flat = pk_ref[:, g * 3 + k, :] - g * HW
            yc.append((flat >> 9).astype(jnp.float32) - float(H // 2))
            xc.append((flat & (W - 1)).astype(jnp.float32) - float(W // 2))
        for b in range(4):
            pts = []
            for k in range(3):
                d = d_ref[:, (g * 4 + b) * 3 + k, :]
                ad = jnp.abs(d)
                pts.append((xc[k] * ad * (1.0 / FX), yc[k] * ad * (1.0 / FY), d))
            e12 = tuple(pts[1][c] - pts[0][c] for c in range(3))
            e13 = tuple(pts[2][c] - pts[0][c] for c in range(3))
            nx = e12[1] * e13[2] - e12[2] * e13[1]
            ny = e12[2] * e13[0] - e12[0] * e13[2]
            nz = e12[0] * e13[1] - e12[1] * e13[0]
            norm = jnp.sqrt(nx * nx + ny * ny + nz * nz)
            denom = norm + jnp.where(norm == 0.0, jnp.float32(0.01),
                                     jnp.float32(0.0))
            normals[g][b] = (nx / denom, ny / denom, nz / denom)

    ji = (lax.broadcasted_iota(jnp.int32, (NTILES, PER_TILE), 0) * PER_TILE
          + lax.broadcasted_iota(jnp.int32, (NTILES, PER_TILE), 1))
    valid = ji < NS

    losses = []
    for b in range(4):
        gt_n, dt_n = normals[0][b], normals[1][b]
        losses.append(sum(jnp.abs(gt_n[c] - dt_n[c]) for c in range(3)))

    inf_bits = jnp.int32(0x7F800000)
    mbits = [jnp.where(valid, lax.bitcast_convert_type(v, jnp.int32), inf_bits)
             for v in losses]

    def bisect(_, lohi):
        lo, hi = lohi
        mid = lo + (hi - lo) // 2
        cnt = sum(jnp.sum((mb <= mid).astype(jnp.int32)) for mb in mbits)
        ge = cnt >= K_DROP
        return jnp.where(ge, lo, mid + 1), jnp.where(ge, mid, hi)

    _, t_bits = lax.fori_loop(0, 31, bisect, (jnp.int32(0), inf_bits))
    t = lax.bitcast_convert_type(t_bits, jnp.float32)

    cnt_lt = jnp.float32(0.0)
    sum_lt = jnp.float32(0.0)
    total = jnp.float32(0.0)
    for b in range(4):
        lt = valid & (losses[b] < t)
        cnt_lt = cnt_lt + jnp.sum(lt.astype(jnp.float32))
        sum_lt = sum_lt + jnp.sum(jnp.where(lt, losses[b], 0.0))
        total = total + jnp.sum(jnp.where(valid, losses[b], 0.0))
    dropped = sum_lt + (jnp.float32(K_DROP) - cnt_lt) * t
    out_ref[0, 0] = (total - dropped) * (1.0 / N_KEEP)


def _tc_loss(d24, pk6):
    return pl.pallas_call(
        _tc_loss_body,
        out_shape=jax.ShapeDtypeStruct((1, 1), jnp.float32),
        out_specs=pl.BlockSpec(memory_space=pltpu.SMEM),
    )(d24, pk6)


def kernel(gt_depth, pred_depth, gt_index, pred_index):
    depth_flat = jnp.concatenate(
        [gt_depth.reshape(4, HW), pred_depth.reshape(4, HW)]
    ).reshape(8 * HW)
    idx4 = jnp.concatenate(
        [gt_index.astype(jnp.int32), pred_index.astype(jnp.int32)]
    )
    idx4 = jnp.pad(idx4, ((0, 0), (0, PADN - NUM))).reshape(4, NROWS, 128)
    packed = _prepack(idx4).reshape(2 * PADN)
    d_out, pk_out = _sc_gather(depth_flat, packed, _sample_position_consts())
    out = _tc_loss(d_out.reshape(NTILES, 24, PER_TILE),
                   pk_out.reshape(NTILES, 6, PER_TILE))
    return out.reshape(())


# speedup vs baseline: 6.9342x; 1.4605x over previous
import functools

import jax
import jax.numpy as jnp
from jax import lax
from jax.experimental import pallas as pl
from jax.experimental.pallas import tpu as pltpu
from jax.experimental.pallas import tpu_sc as plsc

H, W = 384, 512
FX, FY = 518.8579, 519.4696
HW = H * W
NUM = 100000
PADN = 100096
NROWS = PADN // 128
NS = 15000
NS_PAD = 16384
PER_TILE = NS_PAD // 32
K_DROP = 15000
N_KEEP = 4 * NS - K_DROP
NTILES = 32


def _sample_position_consts():
    skey = jax.random.key(42)
    sk1, sk2, sk3 = jax.random.split(skey, 3)
    ps = [jnp.pad(jax.random.randint(sk, (NS,), 0, NUM), (0, NS_PAD - NS))
          for sk in (sk1, sk2, sk3)]
    rows = []
    for g in range(2):
        for k in range(3):
            rows.append(ps[k] + g * PADN)
    pidx = jnp.stack(rows).astype(jnp.int32)
    return (pidx.reshape(6, NTILES, PER_TILE)
            .transpose(1, 0, 2).reshape(NTILES, 6 * PER_TILE))


def _prepack_body(idx_ref, out_ref):
    for g in range(2):
        y = idx_ref[2 * g]
        x = idx_ref[2 * g + 1]
        out_ref[g] = y * W + x + g * HW


def _prepack(idx4):
    return pl.pallas_call(
        _prepack_body,
        out_shape=jax.ShapeDtypeStruct((2, NROWS, 128), jnp.int32),
    )(idx4)


def _sc_gather(depth_flat, packed, pidx):
    mesh = plsc.VectorSubcoreMesh(
        core_axis_name="c", subcore_axis_name="s", num_cores=2, num_subcores=16
    )
    half = 3 * PER_TILE

    @functools.partial(
        pl.kernel,
        out_type=(
            jax.ShapeDtypeStruct((NTILES, 24 * PER_TILE), jnp.float32),
            jax.ShapeDtypeStruct((NTILES, 6 * PER_TILE), jnp.int32),
        ),
        mesh=mesh,
        compiler_params=pltpu.CompilerParams(use_tc_tiling_on_sc=False),
        scratch_types=[
            pltpu.VMEM((6 * PER_TILE,), jnp.int32),
            pltpu.VMEM((6 * PER_TILE,), jnp.int32),
            pltpu.VMEM((24 * PER_TILE,), jnp.int32),
            pltpu.VMEM((24 * PER_TILE,), jnp.float32),
            pltpu.VMEM_SHARED((8 * HW,), jnp.float32),
            pltpu.SemaphoreType.DMA,
            pltpu.SemaphoreType.DMA,
            pltpu.SemaphoreType.DMA,
            pltpu.SemaphoreType.DMA,
        ],
    )
    def sc_kernel(depth_hbm, packed_hbm, pidx_hbm, d_out, pk_out,
                  pidx_v, pk_v, addr_v, d_v, depth_sh,
                  sem_a0, sem_a1, sem_b, sem_y):
        wid = lax.axis_index("s") * 2 + lax.axis_index("c")
        sid = lax.axis_index("s")
        dsl = 8 * HW // 16
        pltpu.sync_copy(depth_hbm.at[pl.ds(sid * dsl, dsl)],
                        depth_sh.at[pl.ds(sid * dsl, dsl)])
        pltpu.sync_copy(pidx_hbm.at[wid], pidx_v)
        cp_a = [
            pltpu.async_copy(packed_hbm.at[pidx_v.at[pl.ds(g * half, half)]],
                             pk_v.at[pl.ds(g * half, half)],
                             sem_a0 if g == 0 else sem_a1)
            for g in range(2)
        ]
        plsc.subcore_barrier()

        def addr_body(g, j):
            for k in range(3):
                pk = pk_v[pl.ds((g * 3 + k) * PER_TILE + j * 16, 16)]
                for b in range(4):
                    addr_v[pl.ds(((g * 4 + b) * 3 + k) * PER_TILE
                                 + j * 16, 16)] = pk + (g * 3 + b) * HW

        cp_b = []
        for g in range(2):
            cp_a[g].wait()
            lax.fori_loop(0, PER_TILE // 16,
                          lambda j, c, g=g: (addr_body(g, j), c)[1], 0)
            for h in range(2):
                sl = pl.ds((g * 2 + h) * 6 * PER_TILE, 6 * PER_TILE)
                cp_b.append(pltpu.async_copy(
                    depth_sh.at[addr_v.at[sl]], d_v.at[sl], sem_b))
        cp_y = pltpu.async_copy(pk_v, pk_out.at[wid], sem_y)
        for cp in cp_b:
            cp.wait()
        pltpu.sync_copy(d_v, d_out.at[wid])
        cp_y.wait()

    return sc_kernel(depth_flat, packed, pidx)


def _tc_loss_body(d_ref, pk_ref, out_ref):
    normals = [[None] * 4, [None] * 4]
    for g in range(2):
        yc, xc = [], []
        for k in range(3):
            flat = pk_ref[:, g * 3 + k, :] - g * HW
            yc.append((flat >> 9).astype(jnp.float32) - float(H // 2))
            xc.append((flat & (W - 1)).astype(jnp.float32) - float(W // 2))
        for b in range(4):
            pts = []
            for k in range(3):
                d = d_ref[:, (g * 4 + b) * 3 + k, :]
                ad = jnp.abs(d)
                pts.append((xc[k] * ad * (1.0 / FX), yc[k] * ad * (1.0 / FY), d))
            e12 = tuple(pts[1][c] - pts[0][c] for c in range(3))
            e13 = tuple(pts[2][c] - pts[0][c] for c in range(3))
            nx = e12[1] * e13[2] - e12[2] * e13[1]
            ny = e12[2] * e13[0] - e12[0] * e13[2]
            nz = e12[0] * e13[1] - e12[1] * e13[0]
            norm = jnp.sqrt(nx * nx + ny * ny + nz * nz)
            denom = norm + jnp.where(norm == 0.0, jnp.float32(0.01),
                                     jnp.float32(0.0))
            normals[g][b] = (nx / denom, ny / denom, nz / denom)

    ji = (lax.broadcasted_iota(jnp.int32, (NTILES, PER_TILE), 0) * PER_TILE
          + lax.broadcasted_iota(jnp.int32, (NTILES, PER_TILE), 1))
    valid = ji < NS

    losses = []
    for b in range(4):
        gt_n, dt_n = normals[0][b], normals[1][b]
        losses.append(sum(jnp.abs(gt_n[c] - dt_n[c]) for c in range(3)))

    inf_bits = jnp.int32(0x7F800000)
    mbits = [jnp.where(valid, lax.bitcast_convert_type(v, jnp.int32), inf_bits)
             for v in losses]

    def bisect(_, lohi):
        lo, hi = lohi
        mid = lo + (hi - lo) // 2
        cnt = sum(jnp.sum((mb <= mid).astype(jnp.int32)) for mb in mbits)
        ge = cnt >= K_DROP
        return jnp.where(ge, lo, mid + 1), jnp.where(ge, mid, hi)

    _, t_bits = lax.fori_loop(0, 31, bisect, (jnp.int32(0), inf_bits))
    t = lax.bitcast_convert_type(t_bits, jnp.float32)

    cnt_lt = jnp.float32(0.0)
    sum_lt = jnp.float32(0.0)
    total = jnp.float32(0.0)
    for b in range(4):
        lt = valid & (losses[b] < t)
        cnt_lt = cnt_lt + jnp.sum(lt.astype(jnp.float32))
        sum_lt = sum_lt + jnp.sum(jnp.where(lt, losses[b], 0.0))
        total = total + jnp.sum(jnp.where(valid, losses[b], 0.0))
    dropped = sum_lt + (jnp.float32(K_DROP) - cnt_lt) * t
    out_ref[0, 0] = (total - dropped) * (1.0 / N_KEEP)


def _tc_loss(d24, pk6):
    return pl.pallas_call(
        _tc_loss_body,
        out_shape=jax.ShapeDtypeStruct((1, 1), jnp.float32),
        out_specs=pl.BlockSpec(memory_space=pltpu.SMEM),
    )(d24, pk6)


def kernel(gt_depth, pred_depth, gt_index, pred_index):
    depth_flat = jnp.concatenate(
        [gt_depth.reshape(4, HW), pred_depth.reshape(4, HW)]
    ).reshape(8 * HW)
    idx4 = jnp.concatenate(
        [gt_index.astype(jnp.int32), pred_index.astype(jnp.int32)]
    )
    idx4 = jnp.pad(idx4, ((0, 0), (0, PADN - NUM))).reshape(4, NROWS, 128)
    packed = _prepack(idx4).reshape(2 * PADN)
    d_out, pk_out = _sc_gather(depth_flat, packed, _sample_position_consts())
    out = _tc_loss(d_out.reshape(NTILES, 24, PER_TILE),
                   pk_out.reshape(NTILES, 6, PER_TILE))
    return out.reshape(())


# speedup vs baseline: 9.3694x; 1.3512x over previous
import functools

import jax
import jax.numpy as jnp
from jax import lax
from jax.experimental import pallas as pl
from jax.experimental.pallas import tpu as pltpu
from jax.experimental.pallas import tpu_sc as plsc

H, W = 384, 512
FX, FY = 518.8579, 519.4696
HW = H * W
NUM = 100000
PADN = 100096
NROWS = PADN // 128
NS = 15000
NS_PAD = 16384
PER_TILE = NS_PAD // 32
K_DROP = 15000
N_KEEP = 4 * NS - K_DROP
NTILES = 32


def _sample_position_consts():
    with jax.ensure_compile_time_eval():
        skey = jax.random.key(42)
        sk1, sk2, sk3 = jax.random.split(skey, 3)
        ps = [jnp.pad(jax.random.randint(sk, (NS,), 0, NUM), (0, NS_PAD - NS))
              for sk in (sk1, sk2, sk3)]
        rows = []
        for g in range(2):
            for k in range(3):
                rows.append(ps[k] + g * PADN)
        pidx = jnp.stack(rows).astype(jnp.int32)
        return (pidx.reshape(6, NTILES, PER_TILE)
                .transpose(1, 0, 2).reshape(NTILES, 6 * PER_TILE))


def _prepack_body(gt_ref, pred_ref, out_ref):
    for g, ref in ((0, gt_ref), (1, pred_ref)):
        out_ref[g] = ref[0] * W + ref[1] + g * HW


def _prepack(gt_idx, pred_idx):
    return pl.pallas_call(
        _prepack_body,
        out_shape=jax.ShapeDtypeStruct((2, NROWS, 128), jnp.int32),
    )(gt_idx, pred_idx)


def _sc_gather(gt_flat, pred_flat, packed, pidx):
    mesh = plsc.VectorSubcoreMesh(
        core_axis_name="c", subcore_axis_name="s", num_cores=2, num_subcores=16
    )
    half = 3 * PER_TILE

    @functools.partial(
        pl.kernel,
        out_type=(
            jax.ShapeDtypeStruct((NTILES, 24 * PER_TILE), jnp.float32),
            jax.ShapeDtypeStruct((NTILES, 6 * PER_TILE), jnp.int32),
        ),
        mesh=mesh,
        compiler_params=pltpu.CompilerParams(use_tc_tiling_on_sc=False),
        scratch_types=[
            pltpu.VMEM((6 * PER_TILE,), jnp.int32),
            pltpu.VMEM((6 * PER_TILE,), jnp.int32),
            pltpu.VMEM((24 * PER_TILE,), jnp.int32),
            pltpu.VMEM((24 * PER_TILE,), jnp.float32),
            pltpu.VMEM_SHARED((8 * HW,), jnp.float32),
            pltpu.SemaphoreType.DMA,
            pltpu.SemaphoreType.DMA,
            pltpu.SemaphoreType.DMA,
            pltpu.SemaphoreType.DMA,
        ],
    )
    def sc_kernel(gt_hbm, pred_hbm, packed_hbm, pidx_hbm, d_out, pk_out,
                  pidx_v, pk_v, addr_v, d_v, depth_sh,
                  sem_a0, sem_a1, sem_b, sem_y):
        wid = lax.axis_index("s") * 2 + lax.axis_index("c")
        sid = lax.axis_index("s")
        dsl = 4 * HW // 16
        pltpu.sync_copy(gt_hbm.at[pl.ds(sid * dsl, dsl)],
                        depth_sh.at[pl.ds(sid * dsl, dsl)])
        pltpu.sync_copy(pred_hbm.at[pl.ds(sid * dsl, dsl)],
                        depth_sh.at[pl.ds(4 * HW + sid * dsl, dsl)])
        pltpu.sync_copy(pidx_hbm.at[wid], pidx_v)
        cp_a = [
            pltpu.async_copy(packed_hbm.at[pidx_v.at[pl.ds(g * half, half)]],
                             pk_v.at[pl.ds(g * half, half)],
                             sem_a0 if g == 0 else sem_a1)
            for g in range(2)
        ]
        plsc.subcore_barrier()

        def addr_body(g, j):
            for k in range(3):
                pk = pk_v[pl.ds((g * 3 + k) * PER_TILE + j * 16, 16)]
                for b in range(4):
                    addr_v[pl.ds(((g * 4 + b) * 3 + k) * PER_TILE
                                 + j * 16, 16)] = pk + (g * 3 + b) * HW

        cp_b = []
        for g in range(2):
            cp_a[g].wait()
            lax.fori_loop(0, PER_TILE // 16,
                          lambda j, c, g=g: (addr_body(g, j), c)[1], 0)
            for h in range(2):
                sl = pl.ds((g * 2 + h) * 6 * PER_TILE, 6 * PER_TILE)
                cp_b.append(pltpu.async_copy(
                    depth_sh.at[addr_v.at[sl]], d_v.at[sl], sem_b))
        cp_y = pltpu.async_copy(pk_v, pk_out.at[wid], sem_y)
        for cp in cp_b:
            cp.wait()
        pltpu.sync_copy(d_v, d_out.at[wid])
        cp_y.wait()

    return sc_kernel(gt_flat, pred_flat, packed, pidx)


def _tc_loss_body(d_ref, pk_ref, out_ref):
    normals = [[None] * 4, [None] * 4]
    for g in range(2):
        yc, xc = [], []
        for k in range(3):
            flat = pk_ref[:, g * 3 + k, :] - g * HW
            yc.append((flat >> 9).astype(jnp.float32) - float(H // 2))
            xc.append((flat & (W - 1)).astype(jnp.float32) - float(W // 2))
        for b in range(4):
            pts = []
            for k in range(3):
                d = d_ref[:, (g * 4 + b) * 3 + k, :]
                ad = jnp.abs(d)
                pts.append((xc[k] * ad * (1.0 / FX), yc[k] * ad * (1.0 / FY), d))
            e12 = tuple(pts[1][c] - pts[0][c] for c in range(3))
            e13 = tuple(pts[2][c] - pts[0][c] for c in range(3))
            nx = e12[1] * e13[2] - e12[2] * e13[1]
            ny = e12[2] * e13[0] - e12[0] * e13[2]
            nz = e12[0] * e13[1] - e12[1] * e13[0]
            norm = jnp.sqrt(nx * nx + ny * ny + nz * nz)
            denom = norm + jnp.where(norm == 0.0, jnp.float32(0.01),
                                     jnp.float32(0.0))
            normals[g][b] = (nx / denom, ny / denom, nz / denom)

    ji = (lax.broadcasted_iota(jnp.int32, (NTILES, PER_TILE), 0) * PER_TILE
          + lax.broadcasted_iota(jnp.int32, (NTILES, PER_TILE), 1))
    valid = ji < NS

    losses = []
    for b in range(4):
        gt_n, dt_n = normals[0][b], normals[1][b]
        losses.append(sum(jnp.abs(gt_n[c] - dt_n[c]) for c in range(3)))

    inf_bits = jnp.int32(0x7F800000)
    mbits = [jnp.where(valid, lax.bitcast_convert_type(v, jnp.int32), inf_bits)
             for v in losses]

    def bisect(_, lohi):
        lo, hi = lohi
        mid = lo + (hi - lo) // 2
        cnt = sum(jnp.sum((mb <= mid).astype(jnp.int32)) for mb in mbits)
        ge = cnt >= K_DROP
        return jnp.where(ge, lo, mid + 1), jnp.where(ge, mid, hi)

    _, t_bits = lax.fori_loop(0, 31, bisect, (jnp.int32(0), inf_bits))
    t = lax.bitcast_convert_type(t_bits, jnp.float32)

    cnt_lt = jnp.float32(0.0)
    sum_lt = jnp.float32(0.0)
    total = jnp.float32(0.0)
    for b in range(4):
        lt = valid & (losses[b] < t)
        cnt_lt = cnt_lt + jnp.sum(lt.astype(jnp.float32))
        sum_lt = sum_lt + jnp.sum(jnp.where(lt, losses[b], 0.0))
        total = total + jnp.sum(jnp.where(valid, losses[b], 0.0))
    dropped = sum_lt + (jnp.float32(K_DROP) - cnt_lt) * t
    out_ref[0, 0] = (total - dropped) * (1.0 / N_KEEP)


def _tc_loss(d24, pk6):
    return pl.pallas_call(
        _tc_loss_body,
        out_shape=jax.ShapeDtypeStruct((1, 1), jnp.float32),
        out_specs=pl.BlockSpec(memory_space=pltpu.SMEM),
    )(d24, pk6)


def kernel(gt_depth, pred_depth, gt_index, pred_index):
    def pad_idx(a):
        a = jnp.pad(a.astype(jnp.int32), ((0, 0), (0, PADN - NUM)))
        return a.reshape(2, NROWS, 128)

    packed = _prepack(pad_idx(gt_index), pad_idx(pred_index)).reshape(2 * PADN)
    d_out, pk_out = _sc_gather(gt_depth.reshape(4 * HW),
                               pred_depth.reshape(4 * HW),
                               packed, _sample_position_consts())
    out = _tc_loss(d_out.reshape(NTILES, 24, PER_TILE),
                   pk_out.reshape(NTILES, 6, PER_TILE))
    return out.reshape(())
